# baseline (speedup 1.0000x reference)
import functools

import jax
import jax.numpy as jnp
from jax import lax
from jax.experimental import pallas as pl
from jax.experimental.pallas import tpu as pltpu
from jax.experimental.pallas import tpu_sc as plsc

VOCAB = 400000
EMBED_DIM = 300
BATCH = 16384

_NUM_CORES = 2
_NUM_SUBCORES = 16
_NW = _NUM_CORES * _NUM_SUBCORES
_B_PER_W = BATCH // _NW
_CHUNK = 128
_NCHUNK = _B_PER_W // _CHUNK
_NWIN = 3


@functools.partial(
    pl.kernel,
    mesh=plsc.VectorSubcoreMesh(core_axis_name="c", subcore_axis_name="s"),
    out_type=jax.ShapeDtypeStruct((BATCH, EMBED_DIM), jnp.float32),
    scratch_types=[
        pltpu.VMEM((_NCHUNK, _CHUNK), jnp.int32),
        pltpu.VMEM((2, _NWIN, _CHUNK, 128), jnp.float32),
        pltpu.SemaphoreType.DMA,
        pltpu.SemaphoreType.DMA,
    ],
    compiler_params=pltpu.CompilerParams(disable_bounds_checks=True),
)
def _gather_kernel(idx_hbm, table_hbm, out_hbm, idx_v, rows_v, sem0, sem1):
    wid = lax.axis_index("s") * _NUM_CORES + lax.axis_index("c")
    base = wid * _B_PER_W

    pltpu.sync_copy(idx_hbm.at[wid], idx_v)

    sems = (sem0, sem1)

    wstarts = [jnp.int32(w * 128) for w in range(_NWIN)]

    def start(c, b):
        return [
            pltpu.async_copy(
                table_hbm.at[idx_v.at[c], pl.ds(wstarts[w], 128)],
                rows_v.at[b, w],
                sems[b],
            )
            for w in range(_NWIN)
        ]

    copies = [None, None]
    copies[0] = start(0, 0)
    for c in range(_NCHUNK):
        b = c % 2
        if c + 1 < _NCHUNK:
            copies[1 - b] = start(c + 1, 1 - b)
        for cp in copies[b]:
            cp.wait()
        for w in range(_NWIN):
            pltpu.sync_copy(
                rows_v.at[b, w],
                out_hbm.at[pl.ds(base + c * _CHUNK, _CHUNK),
                           pl.ds(wstarts[w], 128)],
            )


def kernel(class_labels, captions, glove_vectors):
    del class_labels
    idx = captions.reshape(_NW, _NCHUNK, _CHUNK)
    return _gather_kernel(idx, glove_vectors)


# speedup vs baseline: 2.1808x; 2.1808x over previous
import functools

import jax
import jax.numpy as jnp
from jax import lax
from jax.experimental import pallas as pl
from jax.experimental.pallas import tpu as pltpu
from jax.experimental.pallas import tpu_sc as plsc

VOCAB = 400000
EMBED_DIM = 300
BATCH = 16384

_NUM_CORES = 2
_NUM_SUBCORES = 16
_NW = _NUM_CORES * _NUM_SUBCORES
_B_PER_W = BATCH // _NW
_RB = 64
_NB = _B_PER_W // _RB
_NWIN = 3
_NCHUNK = 19


@functools.partial(
    pl.kernel,
    mesh=plsc.VectorSubcoreMesh(core_axis_name="c", subcore_axis_name="s"),
    out_type=jax.ShapeDtypeStruct((BATCH, EMBED_DIM), jnp.float32),
    scratch_types=[
        pltpu.VMEM((_B_PER_W,), jnp.int32),
        pltpu.SMEM((_B_PER_W + 1,), jnp.int32),
        pltpu.VMEM((_NB, _RB), jnp.int32),
        pltpu.VMEM((2, EMBED_DIM, 128), jnp.float32),
        pltpu.VMEM((2, _RB, 384), jnp.float32),
        pltpu.SemaphoreType.DMA,
        pltpu.SemaphoreType.DMA,
    ],
    compiler_params=pltpu.CompilerParams(
        disable_bounds_checks=True, needs_layout_passes=False),
)
def _gather_kernel(idx_hbm, pos_hbm, tab_t_hbm, out_hbm,
                   idx_v, run_s, pos_v, slab_v, row_v,
                   slab_sem, scat_sem):
    wid = lax.axis_index("s") * _NUM_CORES + lax.axis_index("c")

    pltpu.sync_copy(idx_hbm.at[wid], idx_v)
    pltpu.sync_copy(pos_hbm.at[wid], pos_v)

    dim_iota = lax.iota(jnp.int32, 16)
    neg_inf = jnp.int32(-2147483648)

    def getv(j):
        chunk = idx_v[pl.ds((j // 16) * 16, 16)]
        sel = jnp.where(dim_iota == (j % 16), chunk, neg_inf)
        return jnp.max(sel)

    def scan_body(i, carry):
        n, prev_col = carry
        col = getv(i) >> 7
        is_new = col != prev_col

        def record(nn):
            run_s[nn] = i
            return nn + 1

        n = lax.cond(is_new, record, lambda nn: nn, n)
        return n, col

    n_runs, _ = lax.fori_loop(
        0, _B_PER_W, scan_body, (jnp.int32(0), jnp.int32(-1)))
    wstarts = [jnp.int32(w * 128) for w in range(_NWIN)]

    def slab_src(col):
        lane0 = pl.multiple_of(col * 128, 128)
        return tab_t_hbm.at[:, pl.ds(lane0, 128)]

    def start_slab(col, b):
        return pltpu.async_copy(slab_src(col), slab_v.at[b], slab_sem)

    def wait_slab(b):
        pltpu.make_async_copy(slab_src(jnp.int32(0)), slab_v.at[b],
                              slab_sem).wait()

    def fire_batch(bat):
        p = lax.rem(bat, 2)
        for w in range(_NWIN):
            pltpu.async_copy(
                row_v.at[p, :, pl.ds(wstarts[w], 128)],
                out_hbm.at[pos_v.at[bat], pl.ds(wstarts[w], 128)],
                scat_sem,
            )

    def wait_batch(bat):
        p = lax.rem(bat, 2)
        for w in range(_NWIN):
            pltpu.make_async_copy(
                row_v.at[p, :, pl.ds(wstarts[w], 128)],
                out_hbm.at[pos_v.at[bat], pl.ds(wstarts[w], 128)],
                scat_sem,
            ).wait()

    start_slab(getv(jnp.int32(0)) >> 7, 0)

    def run_body(k, _):
        buf = lax.rem(k, 2)
        start = run_s[k]
        end = jnp.where(k + 1 < n_runs,
                        run_s[jnp.minimum(k + 1, _B_PER_W - 1)],
                        jnp.int32(_B_PER_W))

        @pl.when(k + 1 < n_runs)
        def _():
            start_slab(getv(jnp.minimum(end, _B_PER_W - 1)) >> 7, 1 - buf)

        wait_slab(buf)

        def slot_body(j, _):
            lane = jnp.full((16,), getv(j) & 127, jnp.int32)
            p = lax.rem(lax.div(j, _RB), 2)
            r = lax.rem(j, _RB)

            @pl.when(jnp.logical_and(r == 0, j >= 2 * _RB))
            def _():
                wait_batch(lax.div(j, _RB) - 2)

            for m in range(_NCHUNK):
                got = plsc.load_gather(
                    slab_v.at[buf], [dim_iota + (16 * m), lane])
                row_v[p, r, pl.ds(16 * m, 16)] = got

            @pl.when(r == _RB - 1)
            def _():
                fire_batch(lax.div(j, _RB))

            return 0

        lax.fori_loop(start, end, slot_body, 0, unroll=False)
        return 0

    lax.fori_loop(0, n_runs, run_body, 0, unroll=False)
    wait_batch(_NB - 2)
    wait_batch(_NB - 1)


def kernel(class_labels, captions, glove_vectors):
    del class_labels
    sorted_idx, positions = lax.sort_key_val(
        captions, lax.iota(jnp.int32, BATCH))
    idx = sorted_idx.reshape(_NW, _B_PER_W)
    pos = positions.reshape(_NW, _NB, _RB)
    return _gather_kernel(idx, pos, glove_vectors.T)


# speedup vs baseline: 2.2223x; 1.0190x over previous
import functools

import jax
import jax.numpy as jnp
from jax import lax
from jax.experimental import pallas as pl
from jax.experimental.pallas import tpu as pltpu
from jax.experimental.pallas import tpu_sc as plsc

VOCAB = 400000
EMBED_DIM = 300
BATCH = 16384

_NUM_CORES = 2
_NUM_SUBCORES = 16
_NW = _NUM_CORES * _NUM_SUBCORES
_B_PER_W = BATCH // _NW
_RB = 64
_NB = _B_PER_W // _RB
_NWIN = 3
_NCHUNK = 19


@functools.partial(
    pl.kernel,
    mesh=plsc.VectorSubcoreMesh(core_axis_name="c", subcore_axis_name="s"),
    out_type=jax.ShapeDtypeStruct((BATCH, EMBED_DIM), jnp.float32),
    scratch_types=[
        pltpu.VMEM((_B_PER_W,), jnp.int32),
        pltpu.SMEM((_B_PER_W + 1,), jnp.int32),
        pltpu.SMEM((_B_PER_W + 1,), jnp.int32),
        pltpu.VMEM((_NB, _RB), jnp.int32),
        pltpu.VMEM((2, EMBED_DIM, 128), jnp.float32),
        pltpu.VMEM((2, _RB, 384), jnp.float32),
        pltpu.SemaphoreType.DMA,
        pltpu.SemaphoreType.DMA,
    ],
    compiler_params=pltpu.CompilerParams(
        disable_bounds_checks=True, needs_layout_passes=False),
)
def _gather_kernel(idx_hbm, pos_hbm, tab_t_hbm, out_hbm,
                   idx_v, run_s, col_s, pos_v, slab_v, row_v,
                   slab_sem, scat_sem):
    wid = lax.axis_index("s") * _NUM_CORES + lax.axis_index("c")

    pltpu.sync_copy(idx_hbm.at[wid], idx_v)
    pltpu.sync_copy(pos_hbm.at[wid], pos_v)

    dim_iota = lax.iota(jnp.int32, 16)
    neg_inf = jnp.int32(-2147483648)

    def getv(j):
        chunk = idx_v[pl.ds((j // 16) * 16, 16)]
        sel = jnp.where(dim_iota == (j % 16), chunk, neg_inf)
        return jnp.max(sel)

    def splatv(j):
        chunk = idx_v[pl.ds((j // 16) * 16, 16)]
        return chunk.at[jnp.full((16,), j % 16, jnp.int32)].get(
            mode="promise_in_bounds")

    def scan_body(i, carry):
        n, prev_col = carry
        col = getv(i) >> 7
        is_new = col != prev_col

        def record(nn):
            run_s[nn] = i
            col_s[nn] = col
            return nn + 1

        n = lax.cond(is_new, record, lambda nn: nn, n)
        return n, col

    n_runs, _ = lax.fori_loop(
        0, _B_PER_W, scan_body, (jnp.int32(0), jnp.int32(-1)))
    wstarts = [jnp.int32(w * 128) for w in range(_NWIN)]

    def slab_src(col):
        lane0 = pl.multiple_of(col * 128, 128)
        return tab_t_hbm.at[:, pl.ds(lane0, 128)]

    def start_slab(col, b):
        return pltpu.async_copy(slab_src(col), slab_v.at[b], slab_sem)

    def wait_slab(b):
        pltpu.make_async_copy(slab_src(jnp.int32(0)), slab_v.at[b],
                              slab_sem).wait()

    def fire_batch(bat):
        p = lax.rem(bat, 2)
        for w in range(_NWIN):
            pltpu.async_copy(
                row_v.at[p, :, pl.ds(wstarts[w], 128)],
                out_hbm.at[pos_v.at[bat], pl.ds(wstarts[w], 128)],
                scat_sem,
            )

    def wait_batch(bat):
        p = lax.rem(bat, 2)
        for w in range(_NWIN):
            pltpu.make_async_copy(
                row_v.at[p, :, pl.ds(wstarts[w], 128)],
                out_hbm.at[pos_v.at[bat], pl.ds(wstarts[w], 128)],
                scat_sem,
            ).wait()

    start_slab(col_s[0], 0)

    def run_body(k, _):
        buf = lax.rem(k, 2)
        start = run_s[k]
        end = jnp.where(k + 1 < n_runs,
                        run_s[jnp.minimum(k + 1, _B_PER_W - 1)],
                        jnp.int32(_B_PER_W))

        @pl.when(k + 1 < n_runs)
        def _():
            start_slab(col_s[jnp.minimum(k + 1, _B_PER_W - 1)], 1 - buf)

        wait_slab(buf)

        def slot_body(j, _):
            lane = splatv(j) & 127
            p = lax.rem(lax.div(j, _RB), 2)
            r = lax.rem(j, _RB)

            @pl.when(jnp.logical_and(r == 0, j >= 2 * _RB))
            def _():
                wait_batch(lax.div(j, _RB) - 2)

            for m in range(_NCHUNK):
                got = plsc.load_gather(
                    slab_v.at[buf], [dim_iota + (16 * m), lane])
                row_v[p, r, pl.ds(16 * m, 16)] = got

            @pl.when(r == _RB - 1)
            def _():
                fire_batch(lax.div(j, _RB))

            return 0

        lax.fori_loop(start, end, slot_body, 0, unroll=False)
        return 0

    lax.fori_loop(0, n_runs, run_body, 0, unroll=False)
    wait_batch(_NB - 2)
    wait_batch(_NB - 1)


def kernel(class_labels, captions, glove_vectors):
    del class_labels
    sorted_idx, positions = lax.sort_key_val(
        captions, lax.iota(jnp.int32, BATCH))
    idx = sorted_idx.reshape(_NW, _B_PER_W)
    pos = positions.reshape(_NW, _NB, _RB)
    return _gather_kernel(idx, pos, glove_vectors.T)
